# Initial kernel scaffold; baseline (speedup 1.0000x reference)
import jax
import jax.numpy as jnp
from jax.experimental import pallas as pl


def kernel(x, edge_index, W1, b1, W2, b2):
    raise NotImplementedError("write your pallas kernel here")


# speedup vs baseline: 5.7388x; 5.7388x over previous
import functools

import jax
import jax.numpy as jnp
from jax import lax
from jax.experimental import pallas as pl
from jax.experimental.pallas import tpu as pltpu
from jax.experimental.pallas import tpu_sc as plsc

N = 10000
E = 160000
D = 256
H = 128

NC = 2
NS = 16
LANES = 16

CHUNK = 128
EPAD = 163840
NPAD = 10240
TILE_E = EPAD // NS
TILE_CHUNKS = TILE_E // CHUNK
STRIPE = NPAD // NS
DEG_TILE_E = EPAD // (NC * NS)
DEG_CHUNKS = DEG_TILE_E // CHUNK

def _zero_rows(ref, ncols):
    zeros = jnp.zeros((LANES,), jnp.float32)

    def body(i, _):
        for k in range(ncols // LANES):
            ref[i, pl.ds(k * LANES, LANES)] = zeros
        return 0

    lax.fori_loop(0, CHUNK, body, 0)


def _deg_body(dst_hbm, deg_out, idx_v, ones_v, zbuf, accum, sem):
    c = lax.axis_index("c")
    s = lax.axis_index("s")
    wid = s * NC + c

    _zero_rows(zbuf, CHUNK)
    ones = jnp.ones((LANES,), jnp.float32)

    def fill_ones(i, _):
        for k in range(CHUNK // LANES):
            ones_v[i, pl.ds(k * LANES, LANES)] = ones
        return 0

    lax.fori_loop(0, CHUNK, fill_ones, 0)

    for t in range(STRIPE // CHUNK):
        pltpu.sync_copy(zbuf, accum.at[pl.ds(s * STRIPE + t * CHUNK, CHUNK)])
    plsc.subcore_barrier()

    def chunk_body(j, _):
        base = wid * DEG_TILE_E + j * CHUNK
        pltpu.sync_copy(dst_hbm.at[pl.ds(base, CHUNK)], idx_v)
        pltpu.sync_copy(ones_v, accum.at[idx_v], add=True)
        return 0

    lax.fori_loop(0, DEG_CHUNKS, chunk_body, 0)
    plsc.subcore_barrier()

    pltpu.sync_copy(accum.at[pl.ds(s * STRIPE, STRIPE)],
                    deg_out.at[pl.ds(c * NPAD + s * STRIPE, STRIPE)])


def _segsum_body(y_hbm, src2_hbm, dst_hbm, raw_out,
                 sidx, didx, rows, zbuf, accum, sem):
    c = lax.axis_index("c")
    s = lax.axis_index("s")

    _zero_rows(zbuf, H)
    for t in range(STRIPE // CHUNK):
        pltpu.sync_copy(zbuf, accum.at[pl.ds(s * STRIPE + t * CHUNK, CHUNK)])
    plsc.subcore_barrier()

    def chunk_body(j, _):
        e = s * TILE_E + j * CHUNK
        pltpu.sync_copy(src2_hbm.at[pl.ds(c * EPAD + e, CHUNK)], sidx)
        pltpu.sync_copy(dst_hbm.at[pl.ds(e, CHUNK)], didx)
        pltpu.async_copy(y_hbm.at[sidx], rows, sem).wait()
        pltpu.sync_copy(rows, accum.at[didx], add=True)
        return 0

    lax.fori_loop(0, TILE_CHUNKS, chunk_body, 0)
    plsc.subcore_barrier()

    pltpu.sync_copy(accum.at[pl.ds(s * STRIPE, STRIPE)],
                    raw_out.at[pl.ds(c * NPAD + s * STRIPE, STRIPE)])


@functools.cache
def _sc_kernels():
    mesh = plsc.VectorSubcoreMesh(core_axis_name="c", subcore_axis_name="s",
                                  num_cores=NC, num_subcores=NS)
    deg_kernel = pl.kernel(
        _deg_body,
        out_type=jax.ShapeDtypeStruct((NC * NPAD, CHUNK), jnp.float32),
        mesh=mesh,
        scratch_types=[
            pltpu.VMEM((CHUNK,), jnp.int32),
            pltpu.VMEM((CHUNK, CHUNK), jnp.float32),
            pltpu.VMEM((CHUNK, CHUNK), jnp.float32),
            pltpu.VMEM_SHARED((NPAD, CHUNK), jnp.float32),
            pltpu.SemaphoreType.DMA,
        ],
    )
    segsum_kernel = pl.kernel(
        _segsum_body,
        out_type=jax.ShapeDtypeStruct((NC * NPAD, H), jnp.float32),
        mesh=mesh,
        scratch_types=[
            pltpu.VMEM((CHUNK,), jnp.int32),
            pltpu.VMEM((CHUNK,), jnp.int32),
            pltpu.VMEM((CHUNK, H), jnp.float32),
            pltpu.VMEM((CHUNK, H), jnp.float32),
            pltpu.VMEM_SHARED((NPAD, H), jnp.float32),
            pltpu.SemaphoreType.DMA,
        ],
    )
    return deg_kernel, segsum_kernel


_R = 1000


def _dis_block(d0_ref, d1_ref):
    deg = d0_ref[:, 0:1] + d1_ref[:, 0:1] + 1.0
    return lax.rsqrt(deg)


def _k1_body(x_ref, w_ref, d0_ref, d1_ref, o_ref):
    dis = _dis_block(d0_ref, d1_ref)
    o_ref[:] = jnp.dot(x_ref[:], w_ref[:],
                       preferred_element_type=jnp.float32) * dis


def _k2_body(r0_ref, r1_ref, y0_ref, y1_ref, d0_ref, d1_ref, b_ref, w_ref,
             o_ref):
    dis = _dis_block(d0_ref, d1_ref)
    raw = jnp.concatenate([r0_ref[:], r1_ref[:]], axis=1)
    y = jnp.concatenate([y0_ref[:], y1_ref[:]], axis=1)
    h = jnp.maximum((raw + y) * dis + b_ref[:], 0.0)
    o_ref[:] = jnp.dot(h, w_ref[:], preferred_element_type=jnp.float32) * dis


def _k3_body(r0_ref, r1_ref, y0_ref, y1_ref, d0_ref, d1_ref, b_ref, o_ref):
    dis = _dis_block(d0_ref, d1_ref)
    raw = jnp.concatenate([r0_ref[:], r1_ref[:]], axis=1)
    y = jnp.concatenate([y0_ref[:], y1_ref[:]], axis=1)
    z = (raw + y) * dis + b_ref[:]
    m = jnp.max(z, axis=1, keepdims=True)
    lse = jnp.log(jnp.sum(jnp.exp(z - m), axis=1, keepdims=True)) + m
    o_ref[:] = z - lse


def _row_spec(cols):
    return pl.BlockSpec((_R, cols), lambda i, c: (i, 0))


def _row_spec1(cols):
    return pl.BlockSpec((_R, cols), lambda i: (i, 0))


def kernel(x, edge_index, W1, b1, W2, b2):
    src = edge_index[0]
    dst = edge_index[1]
    npad_e = EPAD - E
    pad_src = jnp.zeros((npad_e,), jnp.int32)
    pad_dst = jnp.full((npad_e,), N, jnp.int32)
    dstp = jnp.concatenate([dst, pad_dst])
    src_p = jnp.concatenate([src, pad_src])
    src2 = jnp.concatenate([src_p, src_p + N])

    deg_kernel, segsum_kernel = _sc_kernels()
    degp = deg_kernel(dstp)
    deg0 = degp[:N]
    deg1 = degp[NPAD:NPAD + N]

    y1 = pl.pallas_call(
        _k1_body,
        grid=(N // _R, NC),
        in_specs=[
            pl.BlockSpec((_R, D), lambda i, c: (i, 0)),
            pl.BlockSpec((D, H), lambda i, c: (0, c)),
            _row_spec(CHUNK),
            _row_spec(CHUNK),
        ],
        out_specs=pl.BlockSpec((_R, H), lambda i, c: (c * (N // _R) + i, 0)),
        out_shape=jax.ShapeDtypeStruct((NC * N, H), jnp.float32),
    )(x, W1, deg0, deg1)

    raw1 = segsum_kernel(y1, src2, dstp)

    y2 = pl.pallas_call(
        _k2_body,
        grid=(N // _R, NC),
        in_specs=[
            _row_spec(H), _row_spec(H), _row_spec(H), _row_spec(H),
            _row_spec(CHUNK), _row_spec(CHUNK),
            pl.BlockSpec((1, D), lambda i, c: (0, 0)),
            pl.BlockSpec((D, H), lambda i, c: (0, c)),
        ],
        out_specs=pl.BlockSpec((_R, H), lambda i, c: (c * (N // _R) + i, 0)),
        out_shape=jax.ShapeDtypeStruct((NC * N, H), jnp.float32),
    )(raw1[:N], raw1[NPAD:NPAD + N], y1[:N], y1[N:], deg0, deg1,
      b1.reshape(1, D), W2)

    raw2 = segsum_kernel(y2, src2, dstp)

    out = pl.pallas_call(
        _k3_body,
        grid=(N // _R,),
        in_specs=[
            _row_spec1(H), _row_spec1(H), _row_spec1(H), _row_spec1(H),
            _row_spec1(CHUNK), _row_spec1(CHUNK),
            pl.BlockSpec((1, D), lambda i: (0, 0)),
        ],
        out_specs=pl.BlockSpec((_R, D), lambda i: (i, 0)),
        out_shape=jax.ShapeDtypeStruct((N, D), jnp.float32),
    )(raw2[:N], raw2[NPAD:NPAD + N], y2[:N], y2[N:], deg0, deg1,
      b2.reshape(1, D))

    return out


# speedup vs baseline: 6.7235x; 1.1716x over previous
import functools

import jax
import jax.numpy as jnp
from jax import lax
from jax.experimental import pallas as pl
from jax.experimental.pallas import tpu as pltpu
from jax.experimental.pallas import tpu_sc as plsc

N = 10000
E = 160000
D = 256
H = 128

NC = 2
NS = 16
LANES = 16

CHUNK = 128
EPAD = 163840
NPAD = 10240
TILE_E = EPAD // NS
TILE_CHUNKS = TILE_E // CHUNK
STRIPE = NPAD // NS
DEG_TILE_E = EPAD // (NC * NS)
DEG_CHUNKS = DEG_TILE_E // CHUNK

def _zero_rows(ref, ncols):
    zeros = jnp.zeros((LANES,), jnp.float32)

    def body(i, _):
        for k in range(ncols // LANES):
            ref[i, pl.ds(k * LANES, LANES)] = zeros
        return 0

    lax.fori_loop(0, CHUNK, body, 0)


def _deg_body(dst_hbm, deg_out, idx_v, ones_v, zbuf, accum, sem):
    c = lax.axis_index("c")
    s = lax.axis_index("s")
    wid = s * NC + c

    _zero_rows(zbuf, CHUNK)
    ones = jnp.ones((LANES,), jnp.float32)

    def fill_ones(i, _):
        for k in range(CHUNK // LANES):
            ones_v[i, pl.ds(k * LANES, LANES)] = ones
        return 0

    lax.fori_loop(0, CHUNK, fill_ones, 0)

    for t in range(STRIPE // CHUNK):
        pltpu.sync_copy(zbuf, accum.at[pl.ds(s * STRIPE + t * CHUNK, CHUNK)])
    plsc.subcore_barrier()

    def chunk_body(j, _):
        base = wid * DEG_TILE_E + j * CHUNK
        pltpu.sync_copy(dst_hbm.at[pl.ds(base, CHUNK)], idx_v)
        pltpu.sync_copy(ones_v, accum.at[idx_v], add=True)
        return 0

    lax.fori_loop(0, DEG_CHUNKS, chunk_body, 0)
    plsc.subcore_barrier()

    pltpu.sync_copy(accum.at[pl.ds(s * STRIPE, STRIPE)],
                    deg_out.at[pl.ds(c * NPAD + s * STRIPE, STRIPE)])


_NSLOT = 1


def _segsum_body(y_hbm, src2_hbm, dst_hbm, raw_out,
                 sidx2d, didx2d, rows, accum, gsems, ssems):
    c = lax.axis_index("c")
    s = lax.axis_index("s")

    _zero_rows(rows.at[0], H)
    for t in range(STRIPE // CHUNK):
        pltpu.sync_copy(rows.at[0],
                        accum.at[pl.ds(s * STRIPE + t * CHUNK, CHUNK)])

    pltpu.sync_copy(
        src2_hbm.at[pl.ds(c * (EPAD // CHUNK) + s * TILE_CHUNKS, TILE_CHUNKS)],
        sidx2d)
    pltpu.sync_copy(dst_hbm.at[pl.ds(s * TILE_CHUNKS, TILE_CHUNKS)], didx2d)
    plsc.subcore_barrier()

    def g_start(b, j):
        pltpu.async_copy(y_hbm.at[sidx2d.at[j]], rows.at[b], gsems.at[b])

    def g_wait(b):
        pltpu.make_async_copy(y_hbm.at[pl.ds(0, CHUNK)], rows.at[b],
                              gsems.at[b]).wait()

    def s_start(b, j):
        pltpu.async_copy(rows.at[b], accum.at[didx2d.at[j]], ssems.at[b],
                         add=True)

    def s_wait(b):
        pltpu.make_async_copy(y_hbm.at[pl.ds(0, CHUNK)], rows.at[b],
                              ssems.at[b]).wait()

    for b in range(_NSLOT):
        g_start(b, b)

    steps = TILE_CHUNKS // _NSLOT - 1

    def step(t, _):
        for b in range(_NSLOT):
            g_wait(b)
            s_start(b, t * _NSLOT + b)
        for b in range(_NSLOT):
            s_wait(b)
            g_start(b, t * _NSLOT + b + _NSLOT)
        return 0

    lax.fori_loop(0, steps, step, 0)

    tail = steps * _NSLOT
    for b in range(_NSLOT):
        g_wait(b)
        s_start(b, tail + b)
    for b in range(_NSLOT):
        s_wait(b)
    plsc.subcore_barrier()

    pltpu.sync_copy(accum.at[pl.ds(s * STRIPE, STRIPE)],
                    raw_out.at[pl.ds(c * NPAD + s * STRIPE, STRIPE)])


@functools.cache
def _sc_kernels():
    mesh = plsc.VectorSubcoreMesh(core_axis_name="c", subcore_axis_name="s",
                                  num_cores=NC, num_subcores=NS)
    deg_kernel = pl.kernel(
        _deg_body,
        out_type=jax.ShapeDtypeStruct((NC * NPAD, CHUNK), jnp.float32),
        mesh=mesh,
        scratch_types=[
            pltpu.VMEM((CHUNK,), jnp.int32),
            pltpu.VMEM((CHUNK, CHUNK), jnp.float32),
            pltpu.VMEM((CHUNK, CHUNK), jnp.float32),
            pltpu.VMEM_SHARED((NPAD, CHUNK), jnp.float32),
            pltpu.SemaphoreType.DMA,
        ],
    )
    segsum_kernel = pl.kernel(
        _segsum_body,
        out_type=jax.ShapeDtypeStruct((NC * NPAD, H), jnp.float32),
        mesh=mesh,
        scratch_types=[
            pltpu.VMEM((TILE_CHUNKS, CHUNK), jnp.int32),
            pltpu.VMEM((TILE_CHUNKS, CHUNK), jnp.int32),
            pltpu.VMEM((_NSLOT, CHUNK, H), jnp.float32),
            pltpu.VMEM_SHARED((NPAD, H), jnp.float32),
            pltpu.SemaphoreType.DMA((_NSLOT,)),
            pltpu.SemaphoreType.DMA((_NSLOT,)),
        ],
    )
    return deg_kernel, segsum_kernel


_R = 1000


def _dis_block(d0_ref, d1_ref):
    deg = d0_ref[:, 0:1] + d1_ref[:, 0:1] + 1.0
    return lax.rsqrt(deg)


def _k1_body(x_ref, w_ref, d0_ref, d1_ref, o_ref):
    dis = _dis_block(d0_ref, d1_ref)
    o_ref[:] = jnp.dot(x_ref[:], w_ref[:],
                       preferred_element_type=jnp.float32) * dis


def _k2_body(r0_ref, r1_ref, y0_ref, y1_ref, d0_ref, d1_ref, b_ref, w_ref,
             o_ref):
    dis = _dis_block(d0_ref, d1_ref)
    raw = jnp.concatenate([r0_ref[:], r1_ref[:]], axis=1)
    y = jnp.concatenate([y0_ref[:], y1_ref[:]], axis=1)
    h = jnp.maximum((raw + y) * dis + b_ref[:], 0.0)
    o_ref[:] = jnp.dot(h, w_ref[:], preferred_element_type=jnp.float32) * dis


def _k3_body(r0_ref, r1_ref, y0_ref, y1_ref, d0_ref, d1_ref, b_ref, o_ref):
    dis = _dis_block(d0_ref, d1_ref)
    raw = jnp.concatenate([r0_ref[:], r1_ref[:]], axis=1)
    y = jnp.concatenate([y0_ref[:], y1_ref[:]], axis=1)
    z = (raw + y) * dis + b_ref[:]
    m = jnp.max(z, axis=1, keepdims=True)
    lse = jnp.log(jnp.sum(jnp.exp(z - m), axis=1, keepdims=True)) + m
    o_ref[:] = z - lse


def _row_spec(cols):
    return pl.BlockSpec((_R, cols), lambda i, c: (i, 0))


def _row_spec1(cols):
    return pl.BlockSpec((_R, cols), lambda i: (i, 0))


def kernel(x, edge_index, W1, b1, W2, b2):
    src = edge_index[0]
    dst = edge_index[1]
    npad_e = EPAD - E
    pad_src = jnp.zeros((npad_e,), jnp.int32)
    pad_dst = jnp.full((npad_e,), N, jnp.int32)
    dstp = jnp.concatenate([dst, pad_dst])
    src_p = jnp.concatenate([src, pad_src])
    src2 = jnp.concatenate([src_p, src_p + N])

    src2_2d = src2.reshape(NC * EPAD // CHUNK, CHUNK)
    dstp_2d = dstp.reshape(EPAD // CHUNK, CHUNK)

    deg_kernel, segsum_kernel = _sc_kernels()
    degp = deg_kernel(dstp)
    deg0 = degp[:N]
    deg1 = degp[NPAD:NPAD + N]

    y1 = pl.pallas_call(
        _k1_body,
        grid=(N // _R, NC),
        in_specs=[
            pl.BlockSpec((_R, D), lambda i, c: (i, 0)),
            pl.BlockSpec((D, H), lambda i, c: (0, c)),
            _row_spec(CHUNK),
            _row_spec(CHUNK),
        ],
        out_specs=pl.BlockSpec((_R, H), lambda i, c: (c * (N // _R) + i, 0)),
        out_shape=jax.ShapeDtypeStruct((NC * N, H), jnp.float32),
    )(x, W1, deg0, deg1)

    raw1 = segsum_kernel(y1, src2_2d, dstp_2d)

    y2 = pl.pallas_call(
        _k2_body,
        grid=(N // _R, NC),
        in_specs=[
            _row_spec(H), _row_spec(H), _row_spec(H), _row_spec(H),
            _row_spec(CHUNK), _row_spec(CHUNK),
            pl.BlockSpec((1, D), lambda i, c: (0, 0)),
            pl.BlockSpec((D, H), lambda i, c: (0, c)),
        ],
        out_specs=pl.BlockSpec((_R, H), lambda i, c: (c * (N // _R) + i, 0)),
        out_shape=jax.ShapeDtypeStruct((NC * N, H), jnp.float32),
    )(raw1[:N], raw1[NPAD:NPAD + N], y1[:N], y1[N:], deg0, deg1,
      b1.reshape(1, D), W2)

    raw2 = segsum_kernel(y2, src2_2d, dstp_2d)

    out = pl.pallas_call(
        _k3_body,
        grid=(N // _R,),
        in_specs=[
            _row_spec1(H), _row_spec1(H), _row_spec1(H), _row_spec1(H),
            _row_spec1(CHUNK), _row_spec1(CHUNK),
            pl.BlockSpec((1, D), lambda i: (0, 0)),
        ],
        out_specs=pl.BlockSpec((_R, D), lambda i: (i, 0)),
        out_shape=jax.ShapeDtypeStruct((N, D), jnp.float32),
    )(raw2[:N], raw2[NPAD:NPAD + N], y2[:N], y2[N:], deg0, deg1,
      b2.reshape(1, D))

    return out
